# Initial kernel scaffold; baseline (speedup 1.0000x reference)
import functools

import jax
import jax.numpy as jnp
from jax import lax
from jax.experimental import pallas as pl
from jax.experimental.pallas import tpu as pltpu
from jax.experimental.pallas import tpu_sc as plsc

_N = 10000
_E = 320000
_G = 128
_L = 128
_NTILES = 32
_B = 100
_NB = _E // _NTILES // _B
_ZR = 104
_RPT = 624
_MR = 312
_MRL = 328


def _sc_agg_body(nchunk, h_hbm, src_hbm, dst_hbm, out_hbm,
                 sidx, didx, rows0, rows1, zbuf, acc, sem0, sem1):
  c = lax.axis_index("c")
  s = lax.axis_index("s")
  w = c * 16 + s

  def _zrow(i, carry):
    for q in range(_L // 16):
      zbuf[i, pl.ds(q * 16, 16)] = jnp.zeros((16,), jnp.float32)
    return carry
  lax.fori_loop(0, _ZR, _zrow, 0)

  pltpu.sync_copy(dst_hbm.at[w], didx)

  for k in range(nchunk):
    row0 = s * _RPT
    for i in range(_RPT // _ZR):
      pltpu.sync_copy(zbuf, acc.at[pl.ds(row0 + i * _ZR, _ZR)])

    @pl.when(s == 15)
    def _():
      pltpu.sync_copy(zbuf.at[pl.ds(0, 16)], acc.at[pl.ds(row0 + _RPT, 16)])

    plsc.subcore_barrier()

    pltpu.sync_copy(src_hbm.at[k * _NTILES + w], sidx)

    pltpu.async_copy(h_hbm.at[sidx.at[0]], rows0, sem0)

    def _pair(it, carry):
      j0 = it * 2
      j1 = j0 + 1
      pltpu.async_copy(h_hbm.at[sidx.at[j1]], rows1, sem1)
      pltpu.make_async_copy(h_hbm.at[sidx.at[j0]], rows0, sem0).wait()
      pltpu.sync_copy(rows0, acc.at[didx.at[j0]], add=True)

      @pl.when(it < _NB // 2 - 1)
      def _():
        pltpu.async_copy(h_hbm.at[sidx.at[j0 + 2]], rows0, sem0)

      pltpu.make_async_copy(h_hbm.at[sidx.at[j1]], rows1, sem1).wait()
      pltpu.sync_copy(rows1, acc.at[didx.at[j1]], add=True)
      return carry

    lax.fori_loop(0, _NB // 2, _pair, 0)

    plsc.subcore_barrier()

    part = k * 2 + c
    out0 = part * _N + s * _RPT

    @pl.when(s < 15)
    def _():
      pltpu.sync_copy(acc.at[pl.ds(s * _RPT, _RPT)],
                      out_hbm.at[pl.ds(out0, _RPT)])

    @pl.when(s == 15)
    def _():
      pltpu.sync_copy(acc.at[pl.ds(s * _RPT, _RPT + 16)],
                      out_hbm.at[pl.ds(out0, _RPT + 16)])


def _make_sc_agg(nchunk):
  mesh = plsc.VectorSubcoreMesh(core_axis_name="c", subcore_axis_name="s")
  return functools.partial(
      pl.kernel,
      mesh=mesh,
      out_type=jax.ShapeDtypeStruct((2 * nchunk * _N, _L), jnp.float32),
      scratch_types=[
          pltpu.VMEM((_NB, _B), jnp.int32),
          pltpu.VMEM((_NB, _B), jnp.int32),
          pltpu.VMEM((_B, _L), jnp.float32),
          pltpu.VMEM((_B, _L), jnp.float32),
          pltpu.VMEM((_ZR, _L), jnp.float32),
          pltpu.VMEM_SHARED((_N, _L), jnp.float32),
          pltpu.SemaphoreType.DMA,
          pltpu.SemaphoreType.DMA,
      ],
  )(functools.partial(_sc_agg_body, nchunk))


_sc_agg1 = _make_sc_agg(1)
_sc_agg2 = _make_sc_agg(2)


def _sc_max_body(h_hbm, b_hbm, out_hbm, hbuf, bbuf, macc, sem):
  c = lax.axis_index("c")
  s = lax.axis_index("s")
  w = c * 16 + s
  start = w * _MR

  def _irow(i, carry):
    for q in range(256 // 16):
      macc[i, pl.ds(q * 16, 16)] = jnp.full((16,), -jnp.inf, jnp.float32)
    return carry
  lax.fori_loop(0, _G, _irow, 0)

  for q in range(352 // 16):
    bbuf[pl.ds(q * 16, 16)] = jnp.full((16,), 2**30, jnp.int32)

  @pl.when(w < 31)
  def _():
    pltpu.sync_copy(b_hbm.at[pl.ds(start, _MR)], bbuf.at[pl.ds(0, _MR)])

  @pl.when(w == 31)
  def _():
    pltpu.sync_copy(b_hbm.at[pl.ds(start, _MRL)], bbuf.at[pl.ds(0, _MRL)])

  for k in range(2):
    @pl.when(w < 31)
    def _():
      pltpu.sync_copy(h_hbm.at[pl.ds(k * _N + start, _MR)],
                      hbuf.at[pl.ds(0, _MR)])

    @pl.when(w == 31)
    def _():
      pltpu.sync_copy(h_hbm.at[pl.ds(k * _N + start, _MRL)],
                      hbuf.at[pl.ds(0, _MRL)])

    def _node(i, carry):
      seg = jnp.min(bbuf[pl.ds(i, 16)], axis=0)
      for q in range(_L // 16):
        col = k * _L + q * 16
        macc[seg, pl.ds(col, 16)] = jnp.maximum(
            macc[seg, pl.ds(col, 16)], hbuf[i, pl.ds(q * 16, 16)])
      return carry

    lax.fori_loop(0, _MR, _node, 0)

    @pl.when(w == 31)
    def _():
      lax.fori_loop(_MR, _MRL, _node, 0)

  pltpu.sync_copy(macc, out_hbm.at[w])


_sc_max = pl.kernel(
    _sc_max_body,
    mesh=plsc.VectorSubcoreMesh(core_axis_name="c", subcore_axis_name="s"),
    out_type=jax.ShapeDtypeStruct((_NTILES, _G, 256), jnp.float32),
    scratch_types=[
        pltpu.VMEM((_MRL, _L), jnp.float32),
        pltpu.VMEM((352,), jnp.int32),
        pltpu.VMEM((_G, 256), jnp.float32),
        pltpu.SemaphoreType.DMA,
    ],
)


def _mlp_body(h_ref, a_ref, wa_ref, wb_ref, o_ref, *, nh, outer_relu):
  hp = h_ref[...]
  ap = a_ref[...]
  zs = [hp[k] + ap[2 * k] + ap[2 * k + 1] for k in range(nh)]
  z = zs[0] if nh == 1 else jnp.concatenate(zs, axis=1)
  y = jnp.maximum(jnp.dot(z, wa_ref[...], preferred_element_type=jnp.float32),
                  0.0)
  y = jnp.dot(y, wb_ref[...], preferred_element_type=jnp.float32)
  if outer_relu:
    y = jnp.maximum(y, 0.0)
  o_ref[0] = y[:, :_L]
  o_ref[1] = y[:, _L:]


def _make_mlp(nh, interpret=False):
  bs = 1000
  din = nh * _L
  return pl.pallas_call(
      functools.partial(_mlp_body, nh=nh, outer_relu=True),
      grid=(_N // bs,),
      in_specs=[
          pl.BlockSpec((nh, bs, _L), lambda i: (0, i, 0)),
          pl.BlockSpec((2 * nh, bs, _L), lambda i: (0, i, 0)),
          pl.BlockSpec((din, 256), lambda i: (0, 0)),
          pl.BlockSpec((256, 256), lambda i: (0, 0)),
      ],
      out_specs=pl.BlockSpec((2, bs, _L), lambda i: (0, i, 0)),
      out_shape=jax.ShapeDtypeStruct((2, _N, _L), jnp.float32),
      interpret=interpret,
  )


_mlp1 = _make_mlp(1)
_mlp2 = _make_mlp(2)


def _head_body(h_ref, b_ref, mx_ref, m1w, m1b, m2w, m2b, ow, ob,
               o_ref, sums, cnts):
  i = pl.program_id(0)
  nblk = pl.num_programs(0) - 1

  @pl.when(i == 0)
  def _():
    sums[...] = jnp.zeros_like(sums)
    cnts[...] = jnp.zeros_like(cnts)

  @pl.when(i < nblk)
  def _():
    h = jnp.concatenate([h_ref[0], h_ref[1]], axis=1)
    seg = b_ref[0, 0, :]
    gids = lax.broadcasted_iota(jnp.int32, (_G, seg.shape[0]), 0)
    onehot = (gids == seg[None, :]).astype(jnp.float32)
    sums[...] += jnp.dot(onehot, h, preferred_element_type=jnp.float32)
    cnts[...] += jnp.sum(onehot, axis=1, keepdims=True)

  @pl.when(i == nblk)
  def _():
    mx = jnp.max(mx_ref[...], axis=0)
    sm = sums[...]
    cnt = jnp.maximum(cnts[...][:, :1], 1.0)
    mean = sm / cnt
    hid = jnp.concatenate([mx, mean, sm], axis=1)
    hid = jnp.maximum(
        jnp.dot(hid, m1w[...], preferred_element_type=jnp.float32) + m1b[...],
        0.0)
    hid = jnp.maximum(
        jnp.dot(hid, m2w[...], preferred_element_type=jnp.float32) + m2b[...],
        0.0)
    res = jnp.dot(hid, ow[...], preferred_element_type=jnp.float32) + ob[...]
    o_ref[...] = jnp.broadcast_to(res, (_G, _L))


def _min_blk(i, hi):
  return jnp.minimum(i, hi)


def _make_head(interpret=False):
  bs = 1000
  nblk = _N // bs
  return pl.pallas_call(
      _head_body,
      grid=(nblk + 1,),
      in_specs=[
          pl.BlockSpec((2, bs, _L), lambda i: (0, _min_blk(i, nblk - 1), 0)),
          pl.BlockSpec((1, 1, bs), lambda i: (_min_blk(i, nblk - 1), 0, 0)),
          pl.BlockSpec((_NTILES, _G, 256), lambda i: (0, 0, 0)),
          pl.BlockSpec((768, 256), lambda i: (0, 0)),
          pl.BlockSpec((1, 256), lambda i: (0, 0)),
          pl.BlockSpec((256, 128), lambda i: (0, 0)),
          pl.BlockSpec((1, 128), lambda i: (0, 0)),
          pl.BlockSpec((128, 1), lambda i: (0, 0)),
          pl.BlockSpec((1, 1), lambda i: (0, 0)),
      ],
      out_specs=pl.BlockSpec((_G, _L), lambda i: (0, 0)),
      out_shape=jax.ShapeDtypeStruct((_G, _L), jnp.float32),
      scratch_shapes=[
          pltpu.VMEM((_G, 256), jnp.float32),
          pltpu.VMEM((_G, _L), jnp.float32),
      ],
      interpret=interpret,
  )


_head = _make_head()


def kernel(x, edge_index, batch_index, W1a, W1b, W2a, W2b, W3a, W3b,
           M1w, M1b, M2w, M2b, Ow, Ob):
  src = edge_index[0].reshape(_NTILES, _NB, _B)
  dst = edge_index[1].reshape(_NTILES, _NB, _B)
  src_l23 = jnp.concatenate([src, src + _N], axis=0)

  agg1 = _sc_agg1(x, src, dst)
  h1p = _mlp1(x[None], agg1.reshape(2, _N, _L), W1a, W1b)

  agg2 = _sc_agg2(h1p.reshape(2 * _N, _L), src_l23, dst)
  h2p = _mlp2(h1p, agg2.reshape(4, _N, _L), W2a, W2b)

  agg3 = _sc_agg2(h2p.reshape(2 * _N, _L), src_l23, dst)
  h3p = _mlp2(h2p, agg3.reshape(4, _N, _L), W3a, W3b)

  maxparts = _sc_max(h3p.reshape(2 * _N, _L), batch_index)

  out = _head(h3p, batch_index.reshape(_N // 1000, 1, 1000), maxparts,
              M1w, M1b.reshape(1, -1), M2w, M2b.reshape(1, -1),
              Ow, Ob.reshape(1, 1))
  return out[:, :1]


# speedup vs baseline: 5.5659x; 5.5659x over previous
import functools

import jax
import jax.numpy as jnp
from jax import lax
from jax.experimental import pallas as pl
from jax.experimental.pallas import tpu as pltpu
from jax.experimental.pallas import tpu_sc as plsc

_N = 10000
_E = 320000
_G = 128
_L = 128
_C = 64
_NTILES = 32
_B = 80
_NB = _E // _NTILES // _B
_ZR = 16
_RPT = 624
_MR = 312
_MRL = 328


def _sc_agg_body(nw, h_hbm, src_hbm, dst_hbm, out_hbm,
                 sidx, didx, rows0, rows1, zbuf, acc, sem0, sem1):
  c = lax.axis_index("c")
  s = lax.axis_index("s")
  w = c * 16 + s

  def _zrow(i, carry):
    for q in range(_C // 16):
      zbuf[i, pl.ds(q * 16, 16)] = jnp.zeros((16,), jnp.float32)
    return carry
  lax.fori_loop(0, _ZR, _zrow, 0)

  pltpu.sync_copy(dst_hbm.at[w], didx)

  for k in range(nw):
    row0 = s * _RPT
    for i in range(_RPT // _ZR):
      pltpu.sync_copy(zbuf, acc.at[pl.ds(row0 + i * _ZR, _ZR)])

    @pl.when(s == 15)
    def _():
      pltpu.sync_copy(zbuf.at[pl.ds(0, 16)], acc.at[pl.ds(row0 + _RPT, 16)])

    plsc.subcore_barrier()

    pltpu.sync_copy(src_hbm.at[k * _NTILES + w], sidx)

    pltpu.async_copy(h_hbm.at[sidx.at[0]], rows0, sem0)

    def _pair(it, carry):
      j0 = it * 2
      j1 = j0 + 1
      pltpu.async_copy(h_hbm.at[sidx.at[j1]], rows1, sem1)
      pltpu.make_async_copy(h_hbm.at[sidx.at[j0]], rows0, sem0).wait()
      pltpu.sync_copy(rows0, acc.at[didx.at[j0]], add=True)

      @pl.when(j0 + 2 < _NB)
      def _():
        pltpu.async_copy(h_hbm.at[sidx.at[j0 + 2]], rows0, sem0)

      pltpu.make_async_copy(h_hbm.at[sidx.at[j1]], rows1, sem1).wait()
      pltpu.sync_copy(rows1, acc.at[didx.at[j1]], add=True)
      return carry

    lax.fori_loop(0, _NB // 2, _pair, 0)

    if _NB % 2:
      jt = _NB - 1
      pltpu.make_async_copy(h_hbm.at[sidx.at[jt]], rows0, sem0).wait()
      pltpu.sync_copy(rows0, acc.at[didx.at[jt]], add=True)

    plsc.subcore_barrier()

    part = k * 2 + c
    out0 = part * _N + s * _RPT

    @pl.when(s < 15)
    def _():
      pltpu.sync_copy(acc.at[pl.ds(s * _RPT, _RPT)],
                      out_hbm.at[pl.ds(out0, _RPT)])

    @pl.when(s == 15)
    def _():
      pltpu.sync_copy(acc.at[pl.ds(s * _RPT, _RPT + 16)],
                      out_hbm.at[pl.ds(out0, _RPT + 16)])


@functools.lru_cache(maxsize=None)
def _make_sc_agg(nw):
  mesh = plsc.VectorSubcoreMesh(core_axis_name="c", subcore_axis_name="s")
  return functools.partial(
      pl.kernel,
      mesh=mesh,
      out_type=jax.ShapeDtypeStruct((2 * nw * _N, _C), jnp.float32),
      scratch_types=[
          pltpu.VMEM((_NB, _B), jnp.int32),
          pltpu.VMEM((_NB, _B), jnp.int32),
          pltpu.VMEM((_B, _C), jnp.float32),
          pltpu.VMEM((_B, _C), jnp.float32),
          pltpu.VMEM((_ZR, _C), jnp.float32),
          pltpu.VMEM_SHARED((_N, _C), jnp.float32),
          pltpu.SemaphoreType.DMA,
          pltpu.SemaphoreType.DMA,
      ],
      compiler_params=pltpu.CompilerParams(use_tc_tiling_on_sc=False),
  )(functools.partial(_sc_agg_body, nw))


def _sc_agg1(*a):
  return _make_sc_agg(2)(*a)


def _sc_agg2(*a):
  return _make_sc_agg(4)(*a)


def _sc_max_body(h_hbm, b_hbm, out_hbm, hbuf, bbuf, macc, sem):
  c = lax.axis_index("c")
  s = lax.axis_index("s")
  w = c * 16 + s
  start = w * _MR

  def _irow(i, carry):
    for q in range(256 // 16):
      macc[i, pl.ds(q * 16, 16)] = jnp.full((16,), -jnp.inf, jnp.float32)
    return carry
  lax.fori_loop(0, _G, _irow, 0)

  for q in range(352 // 16):
    bbuf[pl.ds(q * 16, 16)] = jnp.full((16,), 2**30, jnp.int32)

  @pl.when(w < 31)
  def _():
    pltpu.sync_copy(b_hbm.at[pl.ds(start, _MR)], bbuf.at[pl.ds(0, _MR)])

  @pl.when(w == 31)
  def _():
    pltpu.sync_copy(b_hbm.at[pl.ds(start, _MRL)], bbuf.at[pl.ds(0, _MRL)])

  for k in range(2):
    @pl.when(w < 31)
    def _():
      pltpu.sync_copy(h_hbm.at[pl.ds(k * _N + start, _MR)],
                      hbuf.at[pl.ds(0, _MR)])

    @pl.when(w == 31)
    def _():
      pltpu.sync_copy(h_hbm.at[pl.ds(k * _N + start, _MRL)],
                      hbuf.at[pl.ds(0, _MRL)])

    def _node(i, carry):
      seg = jnp.min(bbuf[pl.ds(i, 16)], axis=0)
      for q in range(_L // 16):
        col = k * _L + q * 16
        macc[seg, pl.ds(col, 16)] = jnp.maximum(
            macc[seg, pl.ds(col, 16)], hbuf[i, pl.ds(q * 16, 16)])
      return carry

    lax.fori_loop(0, _MR, _node, 0)

    @pl.when(w == 31)
    def _():
      lax.fori_loop(_MR, _MRL, _node, 0)

  pltpu.sync_copy(macc, out_hbm.at[w])


@functools.lru_cache(maxsize=None)
def _make_sc_max():
  return pl.kernel(
      _sc_max_body,
      mesh=plsc.VectorSubcoreMesh(core_axis_name="c", subcore_axis_name="s"),
      out_type=jax.ShapeDtypeStruct((_NTILES, _G, 256), jnp.float32),
      scratch_types=[
          pltpu.VMEM((_MRL, _L), jnp.float32),
          pltpu.VMEM((352,), jnp.int32),
          pltpu.VMEM((_G, 256), jnp.float32),
          pltpu.SemaphoreType.DMA,
      ],
      compiler_params=pltpu.CompilerParams(needs_layout_passes=False),
  )


def _sc_max(*a):
  return _make_sc_max()(*a)


def _mlp_body(h_ref, a_ref, wa_ref, wb_ref, o_ref, *, nh, outer_relu):
  hp = h_ref[...]
  ap = a_ref[...]
  zs = []
  for p in range(nh):
    aggp = jnp.concatenate(
        [ap[4 * p] + ap[4 * p + 1], ap[4 * p + 2] + ap[4 * p + 3]], axis=1)
    zs.append(hp[p] + aggp)
  z = zs[0] if nh == 1 else jnp.concatenate(zs, axis=1)
  y = jnp.maximum(jnp.dot(z, wa_ref[...], preferred_element_type=jnp.float32),
                  0.0)
  y = jnp.dot(y, wb_ref[...], preferred_element_type=jnp.float32)
  if outer_relu:
    y = jnp.maximum(y, 0.0)
  o_ref[0] = y[:, :_L]
  o_ref[1] = y[:, _L:]


def _make_mlp(nh, interpret=False):
  bs = 1000
  din = nh * _L
  return pl.pallas_call(
      functools.partial(_mlp_body, nh=nh, outer_relu=True),
      grid=(_N // bs,),
      in_specs=[
          pl.BlockSpec((nh, bs, _L), lambda i: (0, i, 0)),
          pl.BlockSpec((4 * nh, bs, _C), lambda i: (0, i, 0)),
          pl.BlockSpec((din, 256), lambda i: (0, 0)),
          pl.BlockSpec((256, 256), lambda i: (0, 0)),
      ],
      out_specs=pl.BlockSpec((2, bs, _L), lambda i: (0, i, 0)),
      out_shape=jax.ShapeDtypeStruct((2, _N, _L), jnp.float32),
      interpret=interpret,
  )


_mlp1 = _make_mlp(1)
_mlp2 = _make_mlp(2)


def _head_body(h_ref, b_ref, mx_ref, m1w, m1b, m2w, m2b, ow, ob,
               o_ref, sums, cnts):
  i = pl.program_id(0)
  nblk = pl.num_programs(0) - 1

  @pl.when(i == 0)
  def _():
    sums[...] = jnp.zeros_like(sums)
    cnts[...] = jnp.zeros_like(cnts)

  @pl.when(i < nblk)
  def _():
    h = jnp.concatenate([h_ref[0], h_ref[1]], axis=1)
    seg = b_ref[0, 0, :]
    gids = lax.broadcasted_iota(jnp.int32, (_G, seg.shape[0]), 0)
    onehot = (gids == seg[None, :]).astype(jnp.float32)
    sums[...] += jnp.dot(onehot, h, preferred_element_type=jnp.float32)
    cnts[...] += jnp.sum(onehot, axis=1, keepdims=True)

  @pl.when(i == nblk)
  def _():
    mx = jnp.max(mx_ref[...], axis=0)
    sm = sums[...]
    cnt = jnp.maximum(cnts[...][:, :1], 1.0)
    mean = sm / cnt
    hid = jnp.concatenate([mx, mean, sm], axis=1)
    hid = jnp.maximum(
        jnp.dot(hid, m1w[...], preferred_element_type=jnp.float32) + m1b[...],
        0.0)
    hid = jnp.maximum(
        jnp.dot(hid, m2w[...], preferred_element_type=jnp.float32) + m2b[...],
        0.0)
    res = jnp.dot(hid, ow[...], preferred_element_type=jnp.float32) + ob[...]
    o_ref[...] = jnp.broadcast_to(res, (_G, _L))


def _min_blk(i, hi):
  return jnp.minimum(i, hi)


def _make_head(interpret=False):
  bs = 1000
  nblk = _N // bs
  return pl.pallas_call(
      _head_body,
      grid=(nblk + 1,),
      in_specs=[
          pl.BlockSpec((2, bs, _L), lambda i: (0, _min_blk(i, nblk - 1), 0)),
          pl.BlockSpec((1, 1, bs), lambda i: (_min_blk(i, nblk - 1), 0, 0)),
          pl.BlockSpec((_NTILES, _G, 256), lambda i: (0, 0, 0)),
          pl.BlockSpec((768, 256), lambda i: (0, 0)),
          pl.BlockSpec((1, 256), lambda i: (0, 0)),
          pl.BlockSpec((256, 128), lambda i: (0, 0)),
          pl.BlockSpec((1, 128), lambda i: (0, 0)),
          pl.BlockSpec((128, 1), lambda i: (0, 0)),
          pl.BlockSpec((1, 1), lambda i: (0, 0)),
      ],
      out_specs=pl.BlockSpec((_G, _L), lambda i: (0, 0)),
      out_shape=jax.ShapeDtypeStruct((_G, _L), jnp.float32),
      scratch_shapes=[
          pltpu.VMEM((_G, 256), jnp.float32),
          pltpu.VMEM((_G, _L), jnp.float32),
      ],
      interpret=interpret,
  )


_head = _make_head()


def kernel(x, edge_index, batch_index, W1a, W1b, W2a, W2b, W3a, W3b,
           M1w, M1b, M2w, M2b, Ow, Ob):
  src = edge_index[0].reshape(_NTILES, _NB, _B)
  dst = edge_index[1].reshape(_NTILES, _NB, _B)
  s2 = src * 2
  idx1 = jnp.concatenate([s2, s2 + 1], axis=0)
  idx2 = jnp.concatenate(
      [s2, s2 + 1, s2 + 2 * _N, s2 + 2 * _N + 1], axis=0)

  agg1 = _sc_agg1(x.reshape(2 * _N, _C), idx1, dst)
  h1p = _mlp1(x[None], agg1.reshape(4, _N, _C), W1a, W1b)

  agg2 = _sc_agg2(h1p.reshape(4 * _N, _C), idx2, dst)
  h2p = _mlp2(h1p, agg2.reshape(8, _N, _C), W2a, W2b)

  agg3 = _sc_agg2(h2p.reshape(4 * _N, _C), idx2, dst)
  h3p = _mlp2(h2p, agg3.reshape(8, _N, _C), W3a, W3b)

  maxparts = _sc_max(h3p.reshape(2 * _N, _L), batch_index)

  out = _head(h3p, batch_index.reshape(_N // 1000, 1, 1000), maxparts,
              M1w, M1b.reshape(1, -1), M2w, M2b.reshape(1, -1),
              Ow, Ob.reshape(1, 1))
  return out[:, :1]


# speedup vs baseline: 7.3441x; 1.3195x over previous
import functools

import jax
import jax.numpy as jnp
from jax import lax
from jax.experimental import pallas as pl
from jax.experimental.pallas import tpu as pltpu
from jax.experimental.pallas import tpu_sc as plsc

_N = 10000
_E = 320000
_G = 128
_L = 128
_C = 64
_NTILES = 32
_B = 125
_NB = _E // _NTILES // _B
_ZR = 208
_RPT = 624
_MR = 312
_MRL = 328
_RING = 4


def _sc_agg_body(nw, h_hbm, src_hbm, dst_hbm, out_hbm,
                 sidx, didx, rows, zbuf, acc,
                 g0, g1, g2, g3, s0, s1, s2, s3):
  sg = (g0, g1, g2, g3)
  ss = (s0, s1, s2, s3)
  c = lax.axis_index("c")
  s = lax.axis_index("s")
  w = c * 16 + s

  def _zrow(i, carry):
    for q in range(_C // 16):
      zbuf[i, pl.ds(q * 16, 16)] = jnp.zeros((16,), jnp.float32)
    return carry
  lax.fori_loop(0, _ZR, _zrow, 0)

  pltpu.sync_copy(dst_hbm.at[w], didx)

  row0 = s * _RPT
  for k in range(nw):
    pltpu.sync_copy(src_hbm.at[k * _NTILES + w], sidx)
    for b in range(_RING):
      pltpu.async_copy(h_hbm.at[sidx.at[b]], rows.at[b], sg[b])

    for i in range(_RPT // _ZR):
      pltpu.sync_copy(zbuf, acc.at[pl.ds(row0 + i * _ZR, _ZR)])

    @pl.when(s == 15)
    def _():
      pltpu.sync_copy(zbuf.at[pl.ds(0, 16)], acc.at[pl.ds(row0 + _RPT, 16)])

    plsc.subcore_barrier()

    def _ring(it, carry):
      j0 = it * _RING
      for b in range(_RING):
        j = j0 + b
        pltpu.make_async_copy(h_hbm.at[sidx.at[j]], rows.at[b], sg[b]).wait()
        pltpu.async_copy(rows.at[b], acc.at[didx.at[j]], ss[b], add=True)
      for b in range(_RING):
        j = j0 + b
        jn = j + _RING

        @pl.when(jn < _NB)
        def _():
          pltpu.make_async_copy(rows.at[b], acc.at[didx.at[j]], ss[b]).wait()
          pltpu.async_copy(h_hbm.at[sidx.at[jn]], rows.at[b], sg[b])
      return carry

    lax.fori_loop(0, _NB // _RING, _ring, 0)

    for b in range(_RING):
      jt = _NB - _RING + b
      pltpu.make_async_copy(rows.at[b], acc.at[didx.at[jt]], ss[b]).wait()

    plsc.subcore_barrier()

    part = k * 2 + c
    out0 = part * _N + s * _RPT

    @pl.when(s < 15)
    def _():
      pltpu.sync_copy(acc.at[pl.ds(s * _RPT, _RPT)],
                      out_hbm.at[pl.ds(out0, _RPT)])

    @pl.when(s == 15)
    def _():
      pltpu.sync_copy(acc.at[pl.ds(s * _RPT, _RPT + 16)],
                      out_hbm.at[pl.ds(out0, _RPT + 16)])


@functools.lru_cache(maxsize=None)
def _make_sc_agg(nw):
  mesh = plsc.VectorSubcoreMesh(core_axis_name="c", subcore_axis_name="s")
  return functools.partial(
      pl.kernel,
      mesh=mesh,
      out_type=jax.ShapeDtypeStruct((2 * nw * _N, _C), jnp.float32),
      scratch_types=[
          pltpu.VMEM((_NB, _B), jnp.int32),
          pltpu.VMEM((_NB, _B), jnp.int32),
          pltpu.VMEM((_RING, _B, _C), jnp.float32),
          pltpu.VMEM((_ZR, _C), jnp.float32),
          pltpu.VMEM_SHARED((_N, _C), jnp.float32),
      ] + [pltpu.SemaphoreType.DMA] * (2 * _RING),
      compiler_params=pltpu.CompilerParams(use_tc_tiling_on_sc=False),
  )(functools.partial(_sc_agg_body, nw))


def _sc_agg1(*a):
  return _make_sc_agg(2)(*a)


def _sc_agg2(*a):
  return _make_sc_agg(4)(*a)


def _sc_max_body(h_hbm, b_hbm, out_hbm, hbuf, bbuf, macc, sem):
  c = lax.axis_index("c")
  s = lax.axis_index("s")
  w = c * 16 + s
  start = w * _MR

  def _irow(i, carry):
    for q in range(256 // 16):
      macc[i, pl.ds(q * 16, 16)] = jnp.full((16,), -jnp.inf, jnp.float32)
    return carry
  lax.fori_loop(0, _G, _irow, 0)

  for q in range(352 // 16):
    bbuf[pl.ds(q * 16, 16)] = jnp.full((16,), 2**30, jnp.int32)

  @pl.when(w < 31)
  def _():
    pltpu.sync_copy(b_hbm.at[pl.ds(start, _MR)], bbuf.at[pl.ds(0, _MR)])

  @pl.when(w == 31)
  def _():
    pltpu.sync_copy(b_hbm.at[pl.ds(start, _MRL)], bbuf.at[pl.ds(0, _MRL)])

  for k in range(2):
    @pl.when(w < 31)
    def _():
      pltpu.sync_copy(h_hbm.at[pl.ds(k * _N + start, _MR)],
                      hbuf.at[pl.ds(0, _MR)])

    @pl.when(w == 31)
    def _():
      pltpu.sync_copy(h_hbm.at[pl.ds(k * _N + start, _MRL)],
                      hbuf.at[pl.ds(0, _MRL)])

    def _node(i, carry):
      seg = jnp.min(bbuf[pl.ds(i, 16)], axis=0)
      for q in range(_L // 16):
        col = k * _L + q * 16
        macc[seg, pl.ds(col, 16)] = jnp.maximum(
            macc[seg, pl.ds(col, 16)], hbuf[i, pl.ds(q * 16, 16)])
      return carry

    lax.fori_loop(0, _MR, _node, 0)

    @pl.when(w == 31)
    def _():
      lax.fori_loop(_MR, _MRL, _node, 0)

  pltpu.sync_copy(macc, out_hbm.at[w])


@functools.lru_cache(maxsize=None)
def _make_sc_max():
  return pl.kernel(
      _sc_max_body,
      mesh=plsc.VectorSubcoreMesh(core_axis_name="c", subcore_axis_name="s"),
      out_type=jax.ShapeDtypeStruct((_NTILES, _G, 256), jnp.float32),
      scratch_types=[
          pltpu.VMEM((_MRL, _L), jnp.float32),
          pltpu.VMEM((352,), jnp.int32),
          pltpu.VMEM((_G, 256), jnp.float32),
          pltpu.SemaphoreType.DMA,
      ],
      compiler_params=pltpu.CompilerParams(needs_layout_passes=False),
  )


def _sc_max(*a):
  return _make_sc_max()(*a)


def _mlp_body(h_ref, a_ref, wa_ref, wb_ref, o_ref, *, nh, outer_relu):
  hp = h_ref[...]
  ap = a_ref[...]
  zs = []
  for p in range(nh):
    aggp = jnp.concatenate(
        [ap[4 * p] + ap[4 * p + 1], ap[4 * p + 2] + ap[4 * p + 3]], axis=1)
    zs.append(hp[p] + aggp)
  z = zs[0] if nh == 1 else jnp.concatenate(zs, axis=1)
  y = jnp.maximum(jnp.dot(z, wa_ref[...], preferred_element_type=jnp.float32),
                  0.0)
  y = jnp.dot(y, wb_ref[...], preferred_element_type=jnp.float32)
  if outer_relu:
    y = jnp.maximum(y, 0.0)
  o_ref[0] = y[:, :_L]
  o_ref[1] = y[:, _L:]


def _make_mlp(nh, interpret=False):
  bs = 1000
  din = nh * _L
  return pl.pallas_call(
      functools.partial(_mlp_body, nh=nh, outer_relu=True),
      grid=(_N // bs,),
      in_specs=[
          pl.BlockSpec((nh, bs, _L), lambda i: (0, i, 0)),
          pl.BlockSpec((4 * nh, bs, _C), lambda i: (0, i, 0)),
          pl.BlockSpec((din, 256), lambda i: (0, 0)),
          pl.BlockSpec((256, 256), lambda i: (0, 0)),
      ],
      out_specs=pl.BlockSpec((2, bs, _L), lambda i: (0, i, 0)),
      out_shape=jax.ShapeDtypeStruct((2, _N, _L), jnp.float32),
      interpret=interpret,
  )


_mlp1 = _make_mlp(1)
_mlp2 = _make_mlp(2)


def _head_body(h_ref, b_ref, mx_ref, m1w, m1b, m2w, m2b, ow, ob,
               o_ref, sums, cnts):
  i = pl.program_id(0)
  nblk = pl.num_programs(0) - 1

  @pl.when(i == 0)
  def _():
    sums[...] = jnp.zeros_like(sums)
    cnts[...] = jnp.zeros_like(cnts)

  @pl.when(i < nblk)
  def _():
    h = jnp.concatenate([h_ref[0], h_ref[1]], axis=1)
    seg = b_ref[0, 0, :]
    gids = lax.broadcasted_iota(jnp.int32, (_G, seg.shape[0]), 0)
    onehot = (gids == seg[None, :]).astype(jnp.float32)
    sums[...] += jnp.dot(onehot, h, preferred_element_type=jnp.float32)
    cnts[...] += jnp.sum(onehot, axis=1, keepdims=True)

  @pl.when(i == nblk)
  def _():
    mx = jnp.max(mx_ref[...], axis=0)
    sm = sums[...]
    cnt = jnp.maximum(cnts[...][:, :1], 1.0)
    mean = sm / cnt
    hid = jnp.concatenate([mx, mean, sm], axis=1)
    hid = jnp.maximum(
        jnp.dot(hid, m1w[...], preferred_element_type=jnp.float32) + m1b[...],
        0.0)
    hid = jnp.maximum(
        jnp.dot(hid, m2w[...], preferred_element_type=jnp.float32) + m2b[...],
        0.0)
    res = jnp.dot(hid, ow[...], preferred_element_type=jnp.float32) + ob[...]
    o_ref[...] = jnp.broadcast_to(res, (_G, _L))


def _min_blk(i, hi):
  return jnp.minimum(i, hi)


def _make_head(interpret=False):
  bs = 1000
  nblk = _N // bs
  return pl.pallas_call(
      _head_body,
      grid=(nblk + 1,),
      in_specs=[
          pl.BlockSpec((2, bs, _L), lambda i: (0, _min_blk(i, nblk - 1), 0)),
          pl.BlockSpec((1, 1, bs), lambda i: (_min_blk(i, nblk - 1), 0, 0)),
          pl.BlockSpec((_NTILES, _G, 256), lambda i: (0, 0, 0)),
          pl.BlockSpec((768, 256), lambda i: (0, 0)),
          pl.BlockSpec((1, 256), lambda i: (0, 0)),
          pl.BlockSpec((256, 128), lambda i: (0, 0)),
          pl.BlockSpec((1, 128), lambda i: (0, 0)),
          pl.BlockSpec((128, 1), lambda i: (0, 0)),
          pl.BlockSpec((1, 1), lambda i: (0, 0)),
      ],
      out_specs=pl.BlockSpec((_G, _L), lambda i: (0, 0)),
      out_shape=jax.ShapeDtypeStruct((_G, _L), jnp.float32),
      scratch_shapes=[
          pltpu.VMEM((_G, 256), jnp.float32),
          pltpu.VMEM((_G, _L), jnp.float32),
      ],
      interpret=interpret,
  )


_head = _make_head()


def kernel(x, edge_index, batch_index, W1a, W1b, W2a, W2b, W3a, W3b,
           M1w, M1b, M2w, M2b, Ow, Ob):
  src = edge_index[0].reshape(_NTILES, _NB, _B)
  dst = edge_index[1].reshape(_NTILES, _NB, _B)
  s2 = src * 2
  idx1 = jnp.concatenate([s2, s2 + 1], axis=0)
  idx2 = jnp.concatenate(
      [s2, s2 + 1, s2 + 2 * _N, s2 + 2 * _N + 1], axis=0)

  agg1 = _sc_agg1(x.reshape(2 * _N, _C), idx1, dst)
  h1p = _mlp1(x[None], agg1.reshape(4, _N, _C), W1a, W1b)

  agg2 = _sc_agg2(h1p.reshape(4 * _N, _C), idx2, dst)
  h2p = _mlp2(h1p, agg2.reshape(8, _N, _C), W2a, W2b)

  agg3 = _sc_agg2(h2p.reshape(4 * _N, _C), idx2, dst)
  h3p = _mlp2(h2p, agg3.reshape(8, _N, _C), W3a, W3b)

  maxparts = _sc_max(h3p.reshape(2 * _N, _L), batch_index)

  out = _head(h3p, batch_index.reshape(_N // 1000, 1, 1000), maxparts,
              M1w, M1b.reshape(1, -1), M2w, M2b.reshape(1, -1),
              Ow, Ob.reshape(1, 1))
  return out[:, :1]


# speedup vs baseline: 7.3700x; 1.0035x over previous
import functools

import jax
import jax.numpy as jnp
from jax import lax
from jax.experimental import pallas as pl
from jax.experimental.pallas import tpu as pltpu
from jax.experimental.pallas import tpu_sc as plsc

_N = 10000
_E = 320000
_G = 128
_L = 128
_C = 64
_NTILES = 32
_B = 80
_NB = _E // _NTILES // _B
_ZR = 208
_RPT = 624
_MR = 312
_MRL = 328
_RING = 5
_EPT = _E // _NTILES


def _sc_agg_body(nw, h_hbm, src_hbm, dst_hbm, out_hbm,
                 sidx, idxb, didx, rows, zbuf, acc, *sems):
  sg = sems[:_RING]
  ss = sems[_RING:]
  c = lax.axis_index("c")
  s = lax.axis_index("s")
  w = c * 16 + s

  def _zrow(i, carry):
    for q in range(_C // 16):
      zbuf[i, pl.ds(q * 16, 16)] = jnp.zeros((16,), jnp.float32)
    return carry
  lax.fori_loop(0, _ZR, _zrow, 0)

  pltpu.sync_copy(src_hbm.at[w], sidx)
  pltpu.sync_copy(dst_hbm.at[w], didx)

  row0 = s * _RPT
  for k in range(nw):
    off = (k // 2) * 2 * _N + (k % 2)

    def _bld(i, carry):
      v = sidx[pl.ds(i * 16, 16)]
      idxb[pl.ds(i * 16, 16)] = v * 2 + off
      return carry
    lax.fori_loop(0, _EPT // 16, _bld, 0)

    for b in range(_RING):
      pltpu.async_copy(h_hbm.at[idxb.at[pl.ds(b * _B, _B)]],
                       rows.at[b], sg[b])

    for i in range(_RPT // _ZR):
      pltpu.sync_copy(zbuf, acc.at[pl.ds(row0 + i * _ZR, _ZR)])

    @pl.when(s == 15)
    def _():
      pltpu.sync_copy(zbuf.at[pl.ds(0, 16)], acc.at[pl.ds(row0 + _RPT, 16)])

    plsc.subcore_barrier()

    def _ring(it, carry):
      j0 = it * _RING
      for b in range(_RING):
        j = j0 + b
        pltpu.make_async_copy(h_hbm.at[idxb.at[pl.ds(j * _B, _B)]],
                              rows.at[b], sg[b]).wait()
        pltpu.async_copy(rows.at[b], acc.at[didx.at[j]], ss[b], add=True)
      for b in range(_RING):
        j = j0 + b
        jn = j + _RING

        @pl.when(jn < _NB)
        def _():
          pltpu.make_async_copy(rows.at[b], acc.at[didx.at[j]], ss[b]).wait()
          pltpu.async_copy(h_hbm.at[idxb.at[pl.ds(jn * _B, _B)]],
                           rows.at[b], sg[b])
      return carry

    lax.fori_loop(0, _NB // _RING, _ring, 0)

    for b in range(_RING):
      jt = _NB - _RING + b
      pltpu.make_async_copy(rows.at[b], acc.at[didx.at[jt]], ss[b]).wait()

    plsc.subcore_barrier()

    part = k * 2 + c
    out0 = part * _N + s * _RPT

    @pl.when(s < 15)
    def _():
      pltpu.sync_copy(acc.at[pl.ds(s * _RPT, _RPT)],
                      out_hbm.at[pl.ds(out0, _RPT)])

    @pl.when(s == 15)
    def _():
      pltpu.sync_copy(acc.at[pl.ds(s * _RPT, _RPT + 16)],
                      out_hbm.at[pl.ds(out0, _RPT + 16)])


@functools.lru_cache(maxsize=None)
def _make_sc_agg(nw):
  mesh = plsc.VectorSubcoreMesh(core_axis_name="c", subcore_axis_name="s")
  return functools.partial(
      pl.kernel,
      mesh=mesh,
      out_type=jax.ShapeDtypeStruct((2 * nw * _N, _C), jnp.float32),
      scratch_types=[
          pltpu.VMEM((_EPT,), jnp.int32),
          pltpu.VMEM((_EPT,), jnp.int32),
          pltpu.VMEM((_NB, _B), jnp.int32),
          pltpu.VMEM((_RING, _B, _C), jnp.float32),
          pltpu.VMEM((_ZR, _C), jnp.float32),
          pltpu.VMEM_SHARED((_N, _C), jnp.float32),
      ] + [pltpu.SemaphoreType.DMA] * (2 * _RING),
      compiler_params=pltpu.CompilerParams(use_tc_tiling_on_sc=False),
  )(functools.partial(_sc_agg_body, nw))


def _sc_agg1(*a):
  return _make_sc_agg(2)(*a)


def _sc_agg2(*a):
  return _make_sc_agg(4)(*a)


def _sc_max_body(h_hbm, b_hbm, out_hbm, hbuf, bbuf, macc, sem):
  c = lax.axis_index("c")
  s = lax.axis_index("s")
  w = c * 16 + s
  start = w * _MR

  def _irow(i, carry):
    for q in range(256 // 16):
      macc[i, pl.ds(q * 16, 16)] = jnp.full((16,), -jnp.inf, jnp.float32)
    return carry
  lax.fori_loop(0, _G, _irow, 0)

  for q in range(352 // 16):
    bbuf[pl.ds(q * 16, 16)] = jnp.full((16,), 2**30, jnp.int32)

  @pl.when(w < 31)
  def _():
    pltpu.sync_copy(b_hbm.at[pl.ds(start, _MR)], bbuf.at[pl.ds(0, _MR)])

  @pl.when(w == 31)
  def _():
    pltpu.sync_copy(b_hbm.at[pl.ds(start, _MRL)], bbuf.at[pl.ds(0, _MRL)])

  for k in range(2):
    @pl.when(w < 31)
    def _():
      pltpu.sync_copy(h_hbm.at[pl.ds(k * _N + start, _MR)],
                      hbuf.at[pl.ds(0, _MR)])

    @pl.when(w == 31)
    def _():
      pltpu.sync_copy(h_hbm.at[pl.ds(k * _N + start, _MRL)],
                      hbuf.at[pl.ds(0, _MRL)])

    def _node(i, carry):
      seg = jnp.min(bbuf[pl.ds(i, 16)], axis=0)
      for q in range(_L // 16):
        col = k * _L + q * 16
        macc[seg, pl.ds(col, 16)] = jnp.maximum(
            macc[seg, pl.ds(col, 16)], hbuf[i, pl.ds(q * 16, 16)])
      return carry

    lax.fori_loop(0, _MR, _node, 0)

    @pl.when(w == 31)
    def _():
      lax.fori_loop(_MR, _MRL, _node, 0)

  pltpu.sync_copy(macc, out_hbm.at[w])


@functools.lru_cache(maxsize=None)
def _make_sc_max():
  return pl.kernel(
      _sc_max_body,
      mesh=plsc.VectorSubcoreMesh(core_axis_name="c", subcore_axis_name="s"),
      out_type=jax.ShapeDtypeStruct((_NTILES, _G, 256), jnp.float32),
      scratch_types=[
          pltpu.VMEM((_MRL, _L), jnp.float32),
          pltpu.VMEM((352,), jnp.int32),
          pltpu.VMEM((_G, 256), jnp.float32),
          pltpu.SemaphoreType.DMA,
      ],
      compiler_params=pltpu.CompilerParams(needs_layout_passes=False),
  )


def _sc_max(*a):
  return _make_sc_max()(*a)


def _mlp_body(h_ref, a_ref, wa_ref, wb_ref, o_ref, *, nh, outer_relu):
  hp = h_ref[...]
  ap = a_ref[...]
  zs = []
  for p in range(nh):
    aggp = jnp.concatenate(
        [ap[4 * p] + ap[4 * p + 1], ap[4 * p + 2] + ap[4 * p + 3]], axis=1)
    zs.append(hp[p] + aggp)
  z = zs[0] if nh == 1 else jnp.concatenate(zs, axis=1)
  y = jnp.maximum(jnp.dot(z, wa_ref[...], preferred_element_type=jnp.float32),
                  0.0)
  y = jnp.dot(y, wb_ref[...], preferred_element_type=jnp.float32)
  if outer_relu:
    y = jnp.maximum(y, 0.0)
  o_ref[0] = y[:, :_L]
  o_ref[1] = y[:, _L:]


def _make_mlp(nh, interpret=False):
  bs = 1000
  din = nh * _L
  return pl.pallas_call(
      functools.partial(_mlp_body, nh=nh, outer_relu=True),
      grid=(_N // bs,),
      in_specs=[
          pl.BlockSpec((nh, bs, _L), lambda i: (0, i, 0)),
          pl.BlockSpec((4 * nh, bs, _C), lambda i: (0, i, 0)),
          pl.BlockSpec((din, 256), lambda i: (0, 0)),
          pl.BlockSpec((256, 256), lambda i: (0, 0)),
      ],
      out_specs=pl.BlockSpec((2, bs, _L), lambda i: (0, i, 0)),
      out_shape=jax.ShapeDtypeStruct((2, _N, _L), jnp.float32),
      interpret=interpret,
  )


_mlp1 = _make_mlp(1)
_mlp2 = _make_mlp(2)


def _head_sums_body(h_ref, b_ref, o_sums, o_cnts):
  i = pl.program_id(0)

  @pl.when(i == 0)
  def _():
    o_sums[...] = jnp.zeros_like(o_sums)
    o_cnts[...] = jnp.zeros_like(o_cnts)

  h = jnp.concatenate([h_ref[0], h_ref[1]], axis=1)
  seg = b_ref[0, 0, :]
  gids = lax.broadcasted_iota(jnp.int32, (_G, seg.shape[0]), 0)
  onehot = (gids == seg[None, :]).astype(jnp.float32)
  o_sums[...] += jnp.dot(onehot, h, preferred_element_type=jnp.float32)
  o_cnts[...] += jnp.sum(onehot, axis=1, keepdims=True)


def _make_head_sums(interpret=False):
  bs = 1000
  nblk = _N // bs
  return pl.pallas_call(
      _head_sums_body,
      grid=(nblk,),
      in_specs=[
          pl.BlockSpec((2, bs, _L), lambda i: (0, i, 0)),
          pl.BlockSpec((1, 1, bs), lambda i: (i, 0, 0)),
      ],
      out_specs=[
          pl.BlockSpec((_G, 256), lambda i: (0, 0)),
          pl.BlockSpec((_G, _L), lambda i: (0, 0)),
      ],
      out_shape=[
          jax.ShapeDtypeStruct((_G, 256), jnp.float32),
          jax.ShapeDtypeStruct((_G, _L), jnp.float32),
      ],
      interpret=interpret,
  )


def _head_final_body(sums_ref, cnts_ref, mx_ref, m1w, m1b, m2w, m2b, ow, ob,
                     o_ref):
  mx = jnp.max(mx_ref[...], axis=0)
  sm = sums_ref[...]
  cnt = jnp.maximum(cnts_ref[...][:, :1], 1.0)
  mean = sm / cnt
  hid = jnp.concatenate([mx, mean, sm], axis=1)
  hid = jnp.maximum(
      jnp.dot(hid, m1w[...], preferred_element_type=jnp.float32) + m1b[...],
      0.0)
  hid = jnp.maximum(
      jnp.dot(hid, m2w[...], preferred_element_type=jnp.float32) + m2b[...],
      0.0)
  res = jnp.dot(hid, ow[...], preferred_element_type=jnp.float32) + ob[...]
  o_ref[...] = jnp.broadcast_to(res, (_G, _L))


def _make_head_final(interpret=False):
  return pl.pallas_call(
      _head_final_body,
      out_shape=jax.ShapeDtypeStruct((_G, _L), jnp.float32),
      interpret=interpret,
  )


_head_sums = _make_head_sums()
_head_final = _make_head_final()


def kernel(x, edge_index, batch_index, W1a, W1b, W2a, W2b, W3a, W3b,
           M1w, M1b, M2w, M2b, Ow, Ob):
  src = edge_index[0].reshape(_NTILES, _EPT)
  dst = edge_index[1].reshape(_NTILES, _NB, _B)

  agg1 = _sc_agg1(x.reshape(2 * _N, _C), src, dst)
  h1p = _mlp1(x[None], agg1.reshape(4, _N, _C), W1a, W1b)

  agg2 = _sc_agg2(h1p.reshape(4 * _N, _C), src, dst)
  h2p = _mlp2(h1p, agg2.reshape(8, _N, _C), W2a, W2b)

  agg3 = _sc_agg2(h2p.reshape(4 * _N, _C), src, dst)
  h3p = _mlp2(h2p, agg3.reshape(8, _N, _C), W3a, W3b)

  maxparts = _sc_max(h3p.reshape(2 * _N, _L), batch_index)
  sums, cnts = _head_sums(h3p, batch_index.reshape(_N // 1000, 1, 1000))
  out = _head_final(sums, cnts, maxparts,
                    M1w, M1b.reshape(1, -1), M2w, M2b.reshape(1, -1),
                    Ow, Ob.reshape(1, 1))
  return out[:, :1]


# speedup vs baseline: 7.3967x; 1.0036x over previous
import functools

import jax
import jax.numpy as jnp
from jax import lax
from jax.experimental import pallas as pl
from jax.experimental.pallas import tpu as pltpu
from jax.experimental.pallas import tpu_sc as plsc

_N = 10000
_E = 320000
_G = 128
_L = 128
_C = 64
_NTILES = 32
_B = 80
_NB = _E // _NTILES // _B
_ZR = 208
_RPT = 624
_MR = 312
_MRL = 328
_RING = 5
_EPT = _E // _NTILES


def _sc_agg_body(nw, h_hbm, src_hbm, dst_hbm, out_hbm,
                 sidx, idxb, didx, rows, zbuf, acc, *sems):
  sg = sems[:_RING]
  ss = sems[_RING:]
  c = lax.axis_index("c")
  s = lax.axis_index("s")
  w = c * 16 + s

  def _zrow(i, carry):
    for q in range(_C // 16):
      zbuf[i, pl.ds(q * 16, 16)] = jnp.zeros((16,), jnp.float32)
    return carry
  lax.fori_loop(0, _ZR, _zrow, 0)

  pltpu.sync_copy(src_hbm.at[w], sidx)
  pltpu.sync_copy(dst_hbm.at[w], didx)

  row0 = s * _RPT
  for k in range(nw):
    off = (k // 2) * 2 * _N + (k % 2)

    def _bld(i, carry):
      v = sidx[pl.ds(i * 16, 16)]
      idxb[pl.ds(i * 16, 16)] = v * 2 + off
      return carry
    lax.fori_loop(0, _EPT // 16, _bld, 0)

    for b in range(_RING):
      pltpu.async_copy(h_hbm.at[idxb.at[pl.ds(b * _B, _B)]],
                       rows.at[b], sg[b])

    for i in range(_RPT // _ZR):
      pltpu.sync_copy(zbuf, acc.at[pl.ds(row0 + i * _ZR, _ZR)])

    @pl.when(s == 15)
    def _():
      pltpu.sync_copy(zbuf.at[pl.ds(0, 16)], acc.at[pl.ds(row0 + _RPT, 16)])

    plsc.subcore_barrier()

    def _ring(it, carry):
      j0 = it * _RING
      for b in range(_RING):
        j = j0 + b
        pltpu.make_async_copy(h_hbm.at[idxb.at[pl.ds(j * _B, _B)]],
                              rows.at[b], sg[b]).wait()
        pltpu.async_copy(rows.at[b], acc.at[didx.at[j]], ss[b], add=True)
      for b in range(_RING):
        j = j0 + b
        jn = j + _RING

        @pl.when(jn < _NB)
        def _():
          pltpu.make_async_copy(rows.at[b], acc.at[didx.at[j]], ss[b]).wait()
          pltpu.async_copy(h_hbm.at[idxb.at[pl.ds(jn * _B, _B)]],
                           rows.at[b], sg[b])
      return carry

    lax.fori_loop(0, _NB // _RING, _ring, 0)

    for b in range(_RING):
      jt = _NB - _RING + b
      pltpu.make_async_copy(rows.at[b], acc.at[didx.at[jt]], ss[b]).wait()

    plsc.subcore_barrier()

    part = k * 2 + c
    out0 = part * _N + s * _RPT

    @pl.when(s < 15)
    def _():
      pltpu.sync_copy(acc.at[pl.ds(s * _RPT, _RPT)],
                      out_hbm.at[pl.ds(out0, _RPT)])

    @pl.when(s == 15)
    def _():
      pltpu.sync_copy(acc.at[pl.ds(s * _RPT, _RPT + 16)],
                      out_hbm.at[pl.ds(out0, _RPT + 16)])


@functools.lru_cache(maxsize=None)
def _make_sc_agg(nw):
  mesh = plsc.VectorSubcoreMesh(core_axis_name="c", subcore_axis_name="s")
  return functools.partial(
      pl.kernel,
      mesh=mesh,
      out_type=jax.ShapeDtypeStruct((2 * nw * _N, _C), jnp.float32),
      scratch_types=[
          pltpu.VMEM((_EPT,), jnp.int32),
          pltpu.VMEM((_EPT,), jnp.int32),
          pltpu.VMEM((_NB, _B), jnp.int32),
          pltpu.VMEM((_RING, _B, _C), jnp.float32),
          pltpu.VMEM((_ZR, _C), jnp.float32),
          pltpu.VMEM_SHARED((_N, _C), jnp.float32),
      ] + [pltpu.SemaphoreType.DMA] * (2 * _RING),
      compiler_params=pltpu.CompilerParams(use_tc_tiling_on_sc=False),
  )(functools.partial(_sc_agg_body, nw))


def _sc_agg1(*a):
  return _make_sc_agg(2)(*a)


def _sc_agg2(*a):
  return _make_sc_agg(4)(*a)


def _sc_max_body(h_hbm, b_hbm, out_hbm, hbuf, bbuf, macc, sem):
  c = lax.axis_index("c")
  s = lax.axis_index("s")
  w = c * 16 + s
  start = w * _MR

  def _irow(i, carry):
    for q in range(256 // 16):
      macc[i, pl.ds(q * 16, 16)] = jnp.full((16,), -jnp.inf, jnp.float32)
    return carry
  lax.fori_loop(0, _G, _irow, 0)

  for q in range(352 // 16):
    bbuf[pl.ds(q * 16, 16)] = jnp.full((16,), 2**30, jnp.int32)

  @pl.when(w < 31)
  def _():
    pltpu.sync_copy(b_hbm.at[pl.ds(start, _MR)], bbuf.at[pl.ds(0, _MR)])

  @pl.when(w == 31)
  def _():
    pltpu.sync_copy(b_hbm.at[pl.ds(start, _MRL)], bbuf.at[pl.ds(0, _MRL)])

  for k in range(2):
    @pl.when(w < 31)
    def _():
      pltpu.sync_copy(h_hbm.at[pl.ds(k * _N + start, _MR)],
                      hbuf.at[pl.ds(0, _MR)])

    @pl.when(w == 31)
    def _():
      pltpu.sync_copy(h_hbm.at[pl.ds(k * _N + start, _MRL)],
                      hbuf.at[pl.ds(0, _MRL)])

    def _node(i, carry):
      seg = jnp.min(bbuf[pl.ds(i, 16)], axis=0)
      for q in range(_L // 16):
        col = k * _L + q * 16
        macc[seg, pl.ds(col, 16)] = jnp.maximum(
            macc[seg, pl.ds(col, 16)], hbuf[i, pl.ds(q * 16, 16)])
      return carry

    lax.fori_loop(0, _MR, _node, 0)

    @pl.when(w == 31)
    def _():
      lax.fori_loop(_MR, _MRL, _node, 0)

  pltpu.sync_copy(macc, out_hbm.at[w])


@functools.lru_cache(maxsize=None)
def _make_sc_max():
  return pl.kernel(
      _sc_max_body,
      mesh=plsc.VectorSubcoreMesh(core_axis_name="c", subcore_axis_name="s"),
      out_type=jax.ShapeDtypeStruct((_NTILES, _G, 256), jnp.float32),
      scratch_types=[
          pltpu.VMEM((_MRL, _L), jnp.float32),
          pltpu.VMEM((352,), jnp.int32),
          pltpu.VMEM((_G, 256), jnp.float32),
          pltpu.SemaphoreType.DMA,
      ],
      compiler_params=pltpu.CompilerParams(needs_layout_passes=False),
  )


def _sc_max(*a):
  return _make_sc_max()(*a)


def _mlp_body(h_ref, a_ref, wa_ref, wb_ref, o_ref, *, nh, outer_relu):
  hp = h_ref[...]
  ap = a_ref[...]
  zs = []
  for p in range(nh):
    aggp = jnp.concatenate(
        [ap[4 * p] + ap[4 * p + 1], ap[4 * p + 2] + ap[4 * p + 3]], axis=1)
    zs.append(hp[p] + aggp)
  z = zs[0] if nh == 1 else jnp.concatenate(zs, axis=1)
  y = jnp.maximum(jnp.dot(z, wa_ref[...], preferred_element_type=jnp.float32),
                  0.0)
  y = jnp.dot(y, wb_ref[...], preferred_element_type=jnp.float32)
  if outer_relu:
    y = jnp.maximum(y, 0.0)
  o_ref[0] = y[:, :_L]
  o_ref[1] = y[:, _L:]


def _make_mlp(nh, interpret=False):
  bs = 2000
  din = nh * _L
  return pl.pallas_call(
      functools.partial(_mlp_body, nh=nh, outer_relu=True),
      grid=(_N // bs,),
      in_specs=[
          pl.BlockSpec((nh, bs, _L), lambda i: (0, i, 0)),
          pl.BlockSpec((4 * nh, bs, _C), lambda i: (0, i, 0)),
          pl.BlockSpec((din, 256), lambda i: (0, 0)),
          pl.BlockSpec((256, 256), lambda i: (0, 0)),
      ],
      out_specs=pl.BlockSpec((2, bs, _L), lambda i: (0, i, 0)),
      out_shape=jax.ShapeDtypeStruct((2, _N, _L), jnp.float32),
      interpret=interpret,
  )


_mlp1 = _make_mlp(1)
_mlp2 = _make_mlp(2)


def _head_sums_body(h_ref, b_ref, o_sums, o_cnts):
  i = pl.program_id(0)

  @pl.when(i == 0)
  def _():
    o_sums[...] = jnp.zeros_like(o_sums)
    o_cnts[...] = jnp.zeros_like(o_cnts)

  h = jnp.concatenate([h_ref[0], h_ref[1]], axis=1)
  seg = b_ref[0, 0, :]
  gids = lax.broadcasted_iota(jnp.int32, (_G, seg.shape[0]), 0)
  onehot = (gids == seg[None, :]).astype(jnp.float32)
  o_sums[...] += jnp.dot(onehot, h, preferred_element_type=jnp.float32)
  o_cnts[...] += jnp.sum(onehot, axis=1, keepdims=True)


def _make_head_sums(interpret=False):
  bs = 1000
  nblk = _N // bs
  return pl.pallas_call(
      _head_sums_body,
      grid=(nblk,),
      in_specs=[
          pl.BlockSpec((2, bs, _L), lambda i: (0, i, 0)),
          pl.BlockSpec((1, 1, bs), lambda i: (i, 0, 0)),
      ],
      out_specs=[
          pl.BlockSpec((_G, 256), lambda i: (0, 0)),
          pl.BlockSpec((_G, _L), lambda i: (0, 0)),
      ],
      out_shape=[
          jax.ShapeDtypeStruct((_G, 256), jnp.float32),
          jax.ShapeDtypeStruct((_G, _L), jnp.float32),
      ],
      interpret=interpret,
  )


def _head_final_body(sums_ref, cnts_ref, mx_ref, m1w, m1b, m2w, m2b, ow, ob,
                     o_ref):
  mx = jnp.max(mx_ref[...], axis=0)
  sm = sums_ref[...]
  cnt = jnp.maximum(cnts_ref[...][:, :1], 1.0)
  mean = sm / cnt
  hid = jnp.concatenate([mx, mean, sm], axis=1)
  hid = jnp.maximum(
      jnp.dot(hid, m1w[...], preferred_element_type=jnp.float32) + m1b[...],
      0.0)
  hid = jnp.maximum(
      jnp.dot(hid, m2w[...], preferred_element_type=jnp.float32) + m2b[...],
      0.0)
  res = jnp.dot(hid, ow[...], preferred_element_type=jnp.float32) + ob[...]
  o_ref[...] = jnp.broadcast_to(res, (_G, _L))


def _make_head_final(interpret=False):
  return pl.pallas_call(
      _head_final_body,
      out_shape=jax.ShapeDtypeStruct((_G, _L), jnp.float32),
      interpret=interpret,
  )


_head_sums = _make_head_sums()
_head_final = _make_head_final()


def kernel(x, edge_index, batch_index, W1a, W1b, W2a, W2b, W3a, W3b,
           M1w, M1b, M2w, M2b, Ow, Ob):
  src = edge_index[0].reshape(_NTILES, _EPT)
  dst = edge_index[1].reshape(_NTILES, _NB, _B)

  agg1 = _sc_agg1(x.reshape(2 * _N, _C), src, dst)
  h1p = _mlp1(x[None], agg1.reshape(4, _N, _C), W1a, W1b)

  agg2 = _sc_agg2(h1p.reshape(4 * _N, _C), src, dst)
  h2p = _mlp2(h1p, agg2.reshape(8, _N, _C), W2a, W2b)

  agg3 = _sc_agg2(h2p.reshape(4 * _N, _C), src, dst)
  h3p = _mlp2(h2p, agg3.reshape(8, _N, _C), W3a, W3b)

  maxparts = _sc_max(h3p.reshape(2 * _N, _L), batch_index)
  sums, cnts = _head_sums(h3p, batch_index.reshape(_N // 1000, 1, 1000))
  out = _head_final(sums, cnts, maxparts,
                    M1w, M1b.reshape(1, -1), M2w, M2b.reshape(1, -1),
                    Ow, Ob.reshape(1, 1))
  return out[:, :1]
